# baseline (speedup 1.0000x reference)
import jax
import jax.numpy as jnp
from jax import lax
from jax.experimental import pallas as pl
from jax.experimental.pallas import tpu as pltpu
from jax.experimental.pallas import tpu_sc as plsc

DIM = 64
SCALE = 8.0
NC, NS = 2, 16
NW = NC * NS
NBATCH = 4096
SEQ = 200
WB = NBATCH // NW
TW = 128
SPLITS = ((0, 104), (104, 96))


def _emb_body(x_hbm, tab_hbm, z2_hbm, idx_all, r0, r1, r2, r3, semg, semo):
    wid = lax.axis_index("s") * NC + lax.axis_index("c")
    b0 = wid * WB
    flat0 = b0 * SEQ
    pltpu.sync_copy(x_hbm.at[pl.ds(b0, WB)], idx_all)

    def fire_gather(bi, rbuf):
        for off, ln in SPLITS:
            pltpu.make_async_copy(
                tab_hbm.at[idx_all.at[bi, pl.ds(off, ln)]],
                rbuf.at[pl.ds(off, ln)],
                semg,
            ).start()

    def wait_gather(bi, rbuf):
        for off, ln in SPLITS:
            pltpu.make_async_copy(
                tab_hbm.at[idx_all.at[bi, pl.ds(off, ln)]],
                rbuf.at[pl.ds(off, ln)],
                semg,
            ).wait()

    def compute(rbuf):
        def per_row(r, c):
            for c4 in range(DIM // 16):
                sl = pl.ds(c4 * 16, 16)
                rbuf[r, sl] = rbuf[r, sl] * SCALE
            return c

        lax.fori_loop(0, SEQ, per_row, 0)

    def fire_out(bi, rbuf):
        pltpu.make_async_copy(
            rbuf, z2_hbm.at[pl.ds(flat0 + bi * SEQ, SEQ)], semo
        ).start()

    def wait_out(rbuf):
        pltpu.make_async_copy(
            rbuf, z2_hbm.at[pl.ds(flat0, SEQ)], semo
        ).wait()

    bufs = (r0, r1, r2, r3)
    fire_gather(0, r0)
    fire_gather(1, r1)

    def body4(i, c):
        for q in range(4):
            k = 4 * i + q
            buf = bufs[q]
            tgt = bufs[(q + 2) % 4]
            wait_gather(k, buf)
            compute(buf)
            fire_out(k, buf)

            @pl.when(k >= 2)
            def _():
                wait_out(tgt)

            @pl.when(k <= WB - 3)
            def _():
                fire_gather(k + 2, tgt)

        return c

    lax.fori_loop(0, WB // 4, body4, 0)
    wait_out(r2)
    wait_out(r3)


@jax.jit
def kernel(x, table):
    tab_p = jnp.pad(table, ((0, 0), (0, TW - DIM)))
    mesh = plsc.VectorSubcoreMesh(core_axis_name="c", subcore_axis_name="s")
    z2 = pl.kernel(
        _emb_body,
        out_type=jax.ShapeDtypeStruct((NBATCH * SEQ, TW), jnp.float32),
        mesh=mesh,
        compiler_params=pltpu.CompilerParams(
            use_tc_tiling_on_sc=False, needs_layout_passes=False
        ),
        scratch_types=[
            pltpu.VMEM((WB, SEQ), jnp.int32),
            pltpu.VMEM((SEQ, TW), jnp.float32),
            pltpu.VMEM((SEQ, TW), jnp.float32),
            pltpu.VMEM((SEQ, TW), jnp.float32),
            pltpu.VMEM((SEQ, TW), jnp.float32),
            pltpu.SemaphoreType.DMA,
            pltpu.SemaphoreType.DMA,
        ],
    )(x.astype(jnp.int32), tab_p)
    return z2[:, :DIM].reshape(NBATCH, SEQ, DIM)
